# baseline (speedup 1.0000x reference)
import math

import jax
import jax.numpy as jnp
from jax.experimental import pallas as pl

DIM = 2048
RANK = 32
NUM_ENTRIES = 64
_SCALE = 2.0 / math.sqrt(RANK)

_E_BLK = 8
_P = 256


def _adalora_block(idx_ref, s_ref, d_ref, u_ref, o_ref):
    j = pl.program_id(0)
    e0 = j * _E_BLK
    dblk = d_ref[...].astype(jnp.bfloat16).reshape(_E_BLK, DIM, RANK)
    dcat = jnp.concatenate([dblk[e] for e in range(_E_BLK)], axis=1)
    y = jnp.dot(s_ref[...].astype(jnp.bfloat16), dcat,
                preferred_element_type=jnp.float32)
    eid = jax.lax.broadcasted_iota(jnp.int32, (_P, _E_BLK * RANK), 1) // RANK + e0
    keep = eid == idx_ref[...]
    y = jnp.where(keep, y, 0.0) * _SCALE
    z = jnp.dot(y, u_ref[...], preferred_element_type=jnp.float32)

    @pl.when(j == 0)
    def _init():
        o_ref[...] = z

    @pl.when(j > 0)
    def _acc():
        o_ref[...] += z


def kernel(slots, indices, down_proj_values, up_proj_values):
    b, k, d = slots.shape
    p = b * k
    s2 = slots.reshape(p, d)
    idx = indices.reshape(p, 1).astype(jnp.int32)
    dflat = down_proj_values.reshape(NUM_ENTRIES, d * RANK)
    u2 = up_proj_values.reshape(NUM_ENTRIES * RANK, d)

    out = pl.pallas_call(
        _adalora_block,
        grid=(NUM_ENTRIES // _E_BLK,),
        in_specs=[
            pl.BlockSpec((p, 1), lambda j: (0, 0)),
            pl.BlockSpec((p, d), lambda j: (0, 0)),
            pl.BlockSpec((_E_BLK, d * RANK), lambda j: (j, 0)),
            pl.BlockSpec((_E_BLK * RANK, d), lambda j: (j, 0)),
        ],
        out_specs=pl.BlockSpec((p, d), lambda j: (0, 0)),
        out_shape=jax.ShapeDtypeStruct((p, d), jnp.float32),
    )(idx, s2, dflat, u2)
    return out.reshape(b, k, d)


# speedup vs baseline: 2.2528x; 2.2528x over previous
import math

import jax
import jax.numpy as jnp
from jax.experimental import pallas as pl

DIM = 2048
RANK = 32
NUM_ENTRIES = 64
_SCALE = 2.0 / math.sqrt(RANK)

_E_BLK = 8
_P = 256


def _adalora_block(idx_ref, s_ref, d_ref, u_ref, o_ref):
    j = pl.program_id(0)
    e0 = j * _E_BLK
    y = jnp.dot(s_ref[...].astype(jnp.bfloat16), d_ref[...],
                preferred_element_type=jnp.float32)
    eid = jax.lax.broadcasted_iota(jnp.int32, (_P, _E_BLK * RANK), 1) // RANK + e0
    keep = eid == idx_ref[...]
    y = jnp.where(keep, y, 0.0) * _SCALE
    z = jnp.dot(y.astype(jnp.bfloat16), u_ref[...].astype(jnp.bfloat16),
                preferred_element_type=jnp.float32)

    @pl.when(j == 0)
    def _init():
        o_ref[...] = z

    @pl.when(j > 0)
    def _acc():
        o_ref[...] += z


def kernel(slots, indices, down_proj_values, up_proj_values):
    b, k, d = slots.shape
    p = b * k
    s2 = slots.reshape(p, d)
    idx = indices.reshape(p, 1).astype(jnp.int32)
    d2 = jnp.transpose(down_proj_values.astype(jnp.bfloat16), (1, 0, 2)).reshape(
        d, NUM_ENTRIES * RANK)
    u2 = up_proj_values.reshape(NUM_ENTRIES * RANK, d)

    out = pl.pallas_call(
        _adalora_block,
        grid=(NUM_ENTRIES // _E_BLK,),
        in_specs=[
            pl.BlockSpec((p, 1), lambda j: (0, 0)),
            pl.BlockSpec((p, d), lambda j: (0, 0)),
            pl.BlockSpec((d, _E_BLK * RANK), lambda j: (0, j)),
            pl.BlockSpec((_E_BLK * RANK, d), lambda j: (j, 0)),
        ],
        out_specs=pl.BlockSpec((p, d), lambda j: (0, 0)),
        out_shape=jax.ShapeDtypeStruct((p, d), jnp.float32),
    )(idx, s2, d2, u2)
    return out.reshape(b, k, d)


# speedup vs baseline: 2.6740x; 1.1869x over previous
import math

import jax
import jax.numpy as jnp
from jax.experimental import pallas as pl

DIM = 2048
RANK = 32
NUM_ENTRIES = 64
_SCALE = 2.0 / math.sqrt(RANK)

_E_BLK = 16
_P = 256


def _adalora_block(idx_ref, s_ref, d_ref, u_ref, o_ref):
    j = pl.program_id(0)
    e0 = j * _E_BLK
    y = jnp.dot(s_ref[...], d_ref[...], preferred_element_type=jnp.float32)
    eid = jax.lax.broadcasted_iota(jnp.int32, (_P, _E_BLK * RANK), 1) // RANK + e0
    keep = eid == idx_ref[...]
    y = jnp.where(keep, y, 0.0) * _SCALE
    z = jnp.dot(y, u_ref[...], preferred_element_type=jnp.float32)

    @pl.when(j == 0)
    def _init():
        o_ref[...] = z

    @pl.when(j > 0)
    def _acc():
        o_ref[...] += z


def kernel(slots, indices, down_proj_values, up_proj_values):
    b, k, d = slots.shape
    p = b * k
    s2 = slots.reshape(p, d)
    idx = indices.reshape(p, 1).astype(jnp.int32)
    d2 = jnp.transpose(down_proj_values, (1, 0, 2)).reshape(d, NUM_ENTRIES * RANK)
    u2 = up_proj_values.reshape(NUM_ENTRIES * RANK, d)

    out = pl.pallas_call(
        _adalora_block,
        grid=(NUM_ENTRIES // _E_BLK,),
        in_specs=[
            pl.BlockSpec((p, 1), lambda j: (0, 0)),
            pl.BlockSpec((p, d), lambda j: (0, 0)),
            pl.BlockSpec((d, _E_BLK * RANK), lambda j: (0, j)),
            pl.BlockSpec((_E_BLK * RANK, d), lambda j: (j, 0)),
        ],
        out_specs=pl.BlockSpec((p, d), lambda j: (0, 0)),
        out_shape=jax.ShapeDtypeStruct((p, d), jnp.float32),
    )(idx, s2, d2, u2)
    return out.reshape(b, k, d)


# speedup vs baseline: 2.6811x; 1.0026x over previous
import math

import jax
import jax.numpy as jnp
from jax.experimental import pallas as pl

DIM = 2048
RANK = 32
NUM_ENTRIES = 64
_SCALE = 2.0 / math.sqrt(RANK)

_E_BLK = 32
_P = 256


def _adalora_block(idx_ref, s_ref, d_ref, u_ref, o_ref):
    j = pl.program_id(0)
    e0 = j * _E_BLK
    y = jnp.dot(s_ref[...], d_ref[...], preferred_element_type=jnp.float32)
    eid = jax.lax.broadcasted_iota(jnp.int32, (_P, _E_BLK * RANK), 1) // RANK + e0
    keep = eid == idx_ref[...]
    y = jnp.where(keep, y, 0.0) * _SCALE
    z = jnp.dot(y, u_ref[...], preferred_element_type=jnp.float32)

    @pl.when(j == 0)
    def _init():
        o_ref[...] = z

    @pl.when(j > 0)
    def _acc():
        o_ref[...] += z


def kernel(slots, indices, down_proj_values, up_proj_values):
    b, k, d = slots.shape
    p = b * k
    s2 = slots.reshape(p, d)
    idx = indices.reshape(p, 1).astype(jnp.int32)
    d2 = jnp.transpose(down_proj_values, (1, 0, 2)).reshape(d, NUM_ENTRIES * RANK)
    u2 = up_proj_values.reshape(NUM_ENTRIES * RANK, d)

    out = pl.pallas_call(
        _adalora_block,
        grid=(NUM_ENTRIES // _E_BLK,),
        in_specs=[
            pl.BlockSpec((p, 1), lambda j: (0, 0)),
            pl.BlockSpec((p, d), lambda j: (0, 0)),
            pl.BlockSpec((d, _E_BLK * RANK), lambda j: (0, j)),
            pl.BlockSpec((_E_BLK * RANK, d), lambda j: (j, 0)),
        ],
        out_specs=pl.BlockSpec((p, d), lambda j: (0, 0)),
        out_shape=jax.ShapeDtypeStruct((p, d), jnp.float32),
    )(idx, s2, d2, u2)
    return out.reshape(b, k, d)


# speedup vs baseline: 4.0531x; 1.5117x over previous
import math

import jax
import jax.numpy as jnp
from jax.experimental import pallas as pl

DIM = 2048
RANK = 32
NUM_ENTRIES = 64
_SCALE = 2.0 / math.sqrt(RANK)

_E_BLK = 16
_P = 256


def _adalora_block(idx_ref, st_ref, d_ref, u_ref, o_ref):
    j = pl.program_id(0)
    e0 = j * _E_BLK
    yt = jnp.dot(d_ref[...], st_ref[...], preferred_element_type=jnp.float32)
    eid = jax.lax.broadcasted_iota(jnp.int32, (_E_BLK * RANK, _P), 0) // RANK + e0
    keep = eid == idx_ref[...]
    yt = jnp.where(keep, yt, 0.0) * _SCALE
    z = jax.lax.dot_general(yt, u_ref[...], (((0,), (0,)), ((), ())),
                            preferred_element_type=jnp.float32)

    @pl.when(j == 0)
    def _init():
        o_ref[...] = z

    @pl.when(j > 0)
    def _acc():
        o_ref[...] += z


def kernel(slots, indices, down_proj_values, up_proj_values):
    b, k, d = slots.shape
    p = b * k
    st = jnp.transpose(slots.reshape(p, d))
    idx = indices.reshape(1, p).astype(jnp.int32)
    dt = jnp.transpose(down_proj_values, (0, 2, 1)).reshape(NUM_ENTRIES * RANK, d)
    u2 = up_proj_values.reshape(NUM_ENTRIES * RANK, d)

    out = pl.pallas_call(
        _adalora_block,
        grid=(NUM_ENTRIES // _E_BLK,),
        in_specs=[
            pl.BlockSpec((1, p), lambda j: (0, 0)),
            pl.BlockSpec((d, p), lambda j: (0, 0)),
            pl.BlockSpec((_E_BLK * RANK, d), lambda j: (j, 0)),
            pl.BlockSpec((_E_BLK * RANK, d), lambda j: (j, 0)),
        ],
        out_specs=pl.BlockSpec((p, d), lambda j: (0, 0)),
        out_shape=jax.ShapeDtypeStruct((p, d), jnp.float32),
    )(idx, st, dt, u2)
    return out.reshape(b, k, d)
